# baseline (speedup 1.0000x reference)
import functools

import jax
import jax.numpy as jnp
from jax import lax
from jax.experimental import pallas as pl
from jax.experimental.pallas import tpu as pltpu
from jax.experimental.pallas import tpu_sc as plsc

E_DIM = 64
R_DIM = 64
OUT_DIM = E_DIM + R_DIM + E_DIM

_CHUNK = 128


@functools.partial(jax.jit, static_argnames=())
def _run(head, rel, tail, E_emb, R_emb):
    B = head.shape[0]
    info = plsc.get_sparse_core_info()
    NW = info.num_cores * info.num_subcores
    b_per_w = B // NW
    n_chunks = b_per_w // _CHUNK

    idx3 = jnp.stack([head, rel, tail]).astype(jnp.int32)
    idx3 = idx3.reshape(3, NW, n_chunks, _CHUNK)

    mesh = plsc.VectorSubcoreMesh(core_axis_name="c", subcore_axis_name="s")

    @functools.partial(
        pl.kernel,
        out_type=jax.ShapeDtypeStruct((B, OUT_DIM), jnp.float32),
        mesh=mesh,
        scratch_types=[
            pltpu.VMEM((3, n_chunks, _CHUNK), jnp.int32),
            pltpu.VMEM((b_per_w, E_DIM), jnp.float32),
            pltpu.VMEM((b_per_w, R_DIM), jnp.float32),
            pltpu.VMEM((b_per_w, E_DIM), jnp.float32),
            pltpu.SemaphoreType.DMA,
        ],
        compiler_params=pltpu.CompilerParams(use_tc_tiling_on_sc=False),
    )
    def k(e_hbm, r_hbm, idx_hbm, out_hbm, idx_v, h_v, rv_v, t_v, sem):
        wid = lax.axis_index("s") * info.num_cores + lax.axis_index("c")
        base = wid * b_per_w
        pltpu.sync_copy(idx_hbm.at[:, wid], idx_v)
        copies = []
        for j in range(n_chunks):
            rows = pl.ds(j * _CHUNK, _CHUNK)
            copies.append(pltpu.async_copy(
                e_hbm.at[idx_v.at[0, j]], h_v.at[rows], sem))
            copies.append(pltpu.async_copy(
                r_hbm.at[idx_v.at[1, j]], rv_v.at[rows], sem))
            copies.append(pltpu.async_copy(
                e_hbm.at[idx_v.at[2, j]], t_v.at[rows], sem))
        for c in copies:
            c.wait()
        rows = pl.ds(base, b_per_w)
        pltpu.sync_copy(h_v, out_hbm.at[rows, pl.ds(0, E_DIM)])
        pltpu.sync_copy(rv_v, out_hbm.at[rows, pl.ds(E_DIM, R_DIM)])
        pltpu.sync_copy(t_v, out_hbm.at[rows, pl.ds(E_DIM + R_DIM, E_DIM)])

    return k(E_emb, R_emb, idx3)


def kernel(head, rel, tail, E_emb, R_emb):
    return _run(head, rel, tail, E_emb, R_emb)
